# baseline (speedup 1.0000x reference)
import functools

import jax
import jax.numpy as jnp
from jax import lax
from jax.experimental import pallas as pl
from jax.experimental.pallas import tpu as pltpu
from jax.experimental.pallas import tpu_sc as plsc

E = 320000
D = 128
W = D // 2
NC = 2
NS = 16
NW = NC * NS
EPW = E // NW
CH = 128
NCH = -(-EPW // CH)
NPAIR = (NCH + 1) // 2

_mesh = plsc.VectorSubcoreMesh(
    core_axis_name="c", subcore_axis_name="s", num_cores=NC, num_subcores=NS
)


@functools.partial(
    pl.kernel,
    out_type=jax.ShapeDtypeStruct((E,), jnp.float32),
    mesh=_mesh,
    compiler_params=pltpu.CompilerParams(
        needs_layout_passes=False, use_tc_tiling_on_sc=False
    ),
    scratch_types=[
        pltpu.VMEM((EPW,), jnp.int32),
        pltpu.VMEM((EPW,), jnp.int32),
        pltpu.VMEM((EPW,), jnp.float32),
        pltpu.VMEM((2, CH, W), jnp.int32),
        pltpu.VMEM((2, CH, W), jnp.int32),
        pltpu.SemaphoreType.DMA,
        pltpu.SemaphoreType.DMA,
        pltpu.SemaphoreType.DMA,
        pltpu.SemaphoreType.DMA,
    ],
)
def _edge_dot(zu_hbm, zi_hbm, edge_hbm, out_hbm,
              src_v, dst_v, out_v, zu_v, zi_v, su0, si0, su1, si1):
    wid = lax.axis_index("s") * NC + lax.axis_index("c")
    base = wid * EPW
    lane = lax.iota(jnp.int32, 16)
    sems = ((su0, si0), (su1, si1))

    pltpu.sync_copy(edge_hbm.at[0, pl.ds(base, EPW)], src_v)
    pltpu.sync_copy(edge_hbm.at[1, pl.ds(base, EPW)], dst_v)

    def chunk_off(i):
        return jnp.minimum(i * CH, EPW - CH)

    def start(i, b):
        off = chunk_off(i)
        sem_u, sem_i = sems[b]
        pltpu.async_copy(zu_hbm.at[src_v.at[pl.ds(off, CH)]], zu_v.at[b], sem_u)
        pltpu.async_copy(zi_hbm.at[dst_v.at[pl.ds(off, CH)]], zi_v.at[b], sem_i)

    def wait(b):
        sem_u, sem_i = sems[b]
        pltpu.make_async_copy(zu_hbm.at[src_v.at[pl.ds(0, CH)]], zu_v.at[b], sem_u).wait()
        pltpu.make_async_copy(zi_hbm.at[dst_v.at[pl.ds(0, CH)]], zi_v.at[b], sem_i).wait()

    def compute(i, b):
        off = chunk_off(i)
        zu_b = zu_v.at[b]
        zi_b = zi_v.at[b]

        def group_body(g, _):
            e_vec = g * 16 + lane

            def prods(col):
                vu = plsc.bitcast(plsc.load_gather(zu_b, [e_vec, col]), jnp.bfloat16)
                vi = plsc.bitcast(plsc.load_gather(zi_b, [e_vec, col]), jnp.bfloat16)
                return plsc.unpack(vu * vi, format=plsc.PackFormat.INTERLEAVED)

            def w_body(w, carry):
                accs, col = carry
                new = []
                for k in range(4):
                    pa, pb = prods(col)
                    new.append(accs[2 * k] + pa)
                    new.append(accs[2 * k + 1] + pb)
                    col = (col + 1) & (W - 1)
                return tuple(new), col

            z16 = jnp.zeros((16,), jnp.float32)
            accs, _c = lax.fori_loop(
                0, W // 4, w_body, ((z16,) * 8, lane), unroll=16
            )
            acc = ((accs[0] + accs[1]) + (accs[2] + accs[3])) + (
                (accs[4] + accs[5]) + (accs[6] + accs[7]))
            out_v[pl.ds(off + g * 16, 16)] = acc
            return 0

        lax.fori_loop(0, CH // 16, group_body, 0)

    start(0, 0)

    def pair_body(p, _):
        i0 = p * 2
        start(i0 + 1, 1)
        wait(0)
        compute(i0, 0)
        start(i0 + 2, 0)
        wait(1)
        compute(i0 + 1, 1)
        return 0

    lax.fori_loop(0, NPAIR, pair_body, 0)
    wait(0)

    pltpu.sync_copy(out_v, out_hbm.at[pl.ds(base, EPW)])


def _pack_table(z):
    u = lax.bitcast_convert_type(z, jnp.uint32)
    r = (u + 0x8000) >> 16
    packed = r[:, :W] | (r[:, W:] << 16)
    return lax.bitcast_convert_type(packed, jnp.int32)


def kernel(z_user, z_item, edge_index):
    if edge_index.dtype != jnp.int32:
        edge_index = edge_index.astype(jnp.int32)
    return _edge_dot(_pack_table(z_user), _pack_table(z_item), edge_index)


# speedup vs baseline: 2.2816x; 2.2816x over previous
import functools

import jax
import jax.numpy as jnp
from jax import lax
from jax.experimental import pallas as pl
from jax.experimental.pallas import tpu as pltpu
from jax.experimental.pallas import tpu_sc as plsc

E = 320000
D = 128
W = D // 2
NC = 2
NS = 16
NW = NC * NS
EPW = E // NW
CH = 128
NCH = -(-EPW // CH)
NPAIR = (NCH + 1) // 2

_mesh = plsc.VectorSubcoreMesh(
    core_axis_name="c", subcore_axis_name="s", num_cores=NC, num_subcores=NS
)


@functools.partial(
    pl.kernel,
    out_type=jax.ShapeDtypeStruct((E,), jnp.float32),
    mesh=_mesh,
    compiler_params=pltpu.CompilerParams(
        needs_layout_passes=False, use_tc_tiling_on_sc=False
    ),
    scratch_types=[
        pltpu.VMEM((EPW,), jnp.int32),
        pltpu.VMEM((EPW,), jnp.int32),
        pltpu.VMEM((EPW,), jnp.float32),
        pltpu.VMEM((2, CH, W), jnp.int32),
        pltpu.VMEM((2, CH, W), jnp.int32),
        pltpu.SemaphoreType.DMA,
        pltpu.SemaphoreType.DMA,
        pltpu.SemaphoreType.DMA,
        pltpu.SemaphoreType.DMA,
    ],
)
def _edge_dot(zu_hbm, zi_hbm, edge_hbm, out_hbm,
              src_v, dst_v, out_v, zu_v, zi_v, su0, si0, su1, si1):
    wid = lax.axis_index("s") * NC + lax.axis_index("c")
    base = wid * EPW
    lane = lax.iota(jnp.int32, 16)
    sems = ((su0, si0), (su1, si1))

    pltpu.sync_copy(edge_hbm.at[0, pl.ds(base, EPW)], src_v)
    pltpu.sync_copy(edge_hbm.at[1, pl.ds(base, EPW)], dst_v)

    def chunk_off(i):
        return jnp.minimum(i * CH, EPW - CH)

    def start(i, b):
        off = chunk_off(i)
        sem_u, sem_i = sems[b]
        pltpu.async_copy(zu_hbm.at[src_v.at[pl.ds(off, CH)]], zu_v.at[b], sem_u)
        pltpu.async_copy(zi_hbm.at[dst_v.at[pl.ds(off, CH)]], zi_v.at[b], sem_i)

    def wait(b):
        sem_u, sem_i = sems[b]
        pltpu.make_async_copy(zu_hbm.at[src_v.at[pl.ds(0, CH)]], zu_v.at[b], sem_u).wait()
        pltpu.make_async_copy(zi_hbm.at[dst_v.at[pl.ds(0, CH)]], zi_v.at[b], sem_i).wait()

    def compute(i, b):
        off = chunk_off(i)
        zu_b = zu_v.at[b]
        zi_b = zi_v.at[b]

        def group_body(g, _):
            e_vec = g * 16 + lane

            def prods(col):
                vu = plsc.bitcast(plsc.load_gather(zu_b, [e_vec, col]), jnp.bfloat16)
                vi = plsc.bitcast(plsc.load_gather(zi_b, [e_vec, col]), jnp.bfloat16)
                return plsc.unpack(vu * vi, format=plsc.PackFormat.INTERLEAVED)

            def w_body(w, carry):
                accs, col = carry
                new = []
                for k in range(4):
                    pa, pb = prods(col)
                    new.append(accs[2 * k] + pa)
                    new.append(accs[2 * k + 1] + pb)
                    col = (col + 1) & (W - 1)
                return tuple(new), col

            z16 = jnp.zeros((16,), jnp.float32)
            accs, _c = lax.fori_loop(
                0, W // 4, w_body, ((z16,) * 8, lane), unroll=8
            )
            acc = ((accs[0] + accs[1]) + (accs[2] + accs[3])) + (
                (accs[4] + accs[5]) + (accs[6] + accs[7]))
            out_v[pl.ds(off + g * 16, 16)] = acc
            return 0

        lax.fori_loop(0, CH // 16, group_body, 0)

    start(0, 0)

    def pair_body(p, _):
        i0 = p * 2
        start(i0 + 1, 1)
        wait(0)
        compute(i0, 0)
        start(i0 + 2, 0)
        wait(1)
        compute(i0 + 1, 1)
        return 0

    lax.fori_loop(0, NPAIR, pair_body, 0)
    wait(0)

    pltpu.sync_copy(out_v, out_hbm.at[pl.ds(base, EPW)])


def _pack_table(z):
    u = lax.bitcast_convert_type(z, jnp.uint32)
    r = (u + 0x8000) >> 16
    packed = r[:, :W] | (r[:, W:] << 16)
    return lax.bitcast_convert_type(packed, jnp.int32)


def kernel(z_user, z_item, edge_index):
    if edge_index.dtype != jnp.int32:
        edge_index = edge_index.astype(jnp.int32)
    return _edge_dot(_pack_table(z_user), _pack_table(z_item), edge_index)
